# baseline (speedup 1.0000x reference)
import functools

import jax
import jax.numpy as jnp
from jax import lax
from jax.experimental import pallas as pl
from jax.experimental.pallas import tpu as pltpu
from jax.experimental.pallas import tpu_sc as plsc

_B, _S, _H = 8, 2048, 512
_G, _V = 2, 320
_D = 128
_T = _B * _S
_TB = 2048
_NBLK = _T // _TB

_NC, _NS = 2, 16
_NW = _NC * _NS
_ROWS = _T * _G
_RPW = _ROWS // _NW
_CHUNK = 128
_NCH = _RPW // _CHUNK


_LW = 128


def _lane_fold(t):
    w = t.shape[1]
    while w > _LW:
        w //= 2
        t = t[:, :w] + t[:, w:]
    return t


def _argmax_cols(lt):
    m = jnp.max(lt, axis=0, keepdims=True)
    eq = lt == m
    iota = lax.broadcasted_iota(jnp.int32, lt.shape, 0)
    idx = jnp.min(jnp.where(eq, iota, _V), axis=0, keepdims=True)
    cnt = _lane_fold(eq.astype(jnp.float32))
    return idx, cnt


_SPLIT = 4
_SB = _TB // _SPLIT


def _tc_body(x_hbm, w_ref, b_ref, idx_ref, perp_ref, xbuf, s0, s1):
    sems = [s0, s1]

    def stream(k):
        return pltpu.make_async_copy(
            x_hbm.at[pl.ds(k * _TB, _TB), :], xbuf.at[k % 2], sems[k % 2])

    stream(0).start()
    c0 = jnp.zeros((_V, _LW), jnp.float32)
    c1 = jnp.zeros((_V, _LW), jnp.float32)
    for blk in range(_NBLK):
        if blk + 1 < _NBLK:
            stream(blk + 1).start()
        stream(blk).wait()
        parts = []
        for k in range(_SPLIT):
            lt = lax.dot_general(
                w_ref[...], xbuf[blk % 2, pl.ds(k * _SB, _SB), :],
                (((1,), (1,)), ((), ())),
                preferred_element_type=jnp.float32,
                precision=lax.Precision.DEFAULT,
            ) + b_ref[...]
            idx0, p0 = _argmax_cols(lt[:_V])
            idx1, p1 = _argmax_cols(lt[_V:])
            parts.append(jnp.concatenate([idx0, idx1 + _V], axis=0))
            c0 = c0 + p0
            c1 = c1 + p1
        idx_t = jnp.concatenate(parts, axis=1)
        idx_ref[pl.ds(blk * _TB, _TB), :] = jnp.transpose(idx_t, (1, 0))

    p0 = jnp.sum(c0, axis=1, keepdims=True) * (1.0 / _T)
    p1 = jnp.sum(c1, axis=1, keepdims=True) * (1.0 / _T)
    e0 = -jnp.sum(p0 * jnp.log(p0 + 1e-7), keepdims=True)
    e1 = -jnp.sum(p1 * jnp.log(p1 + 1e-7), keepdims=True)
    perp_ref[...] = jnp.exp(e0) + jnp.exp(e1)


@functools.lru_cache(maxsize=1)
def _make_sc_gather():
    @functools.partial(
        pl.kernel,
        mesh=plsc.VectorSubcoreMesh(core_axis_name="c", subcore_axis_name="s"),
        out_type=jax.ShapeDtypeStruct((_ROWS, _D), jnp.float32),
        scratch_types=[
            pltpu.VMEM((_G * _V, _D), jnp.float32),
            pltpu.VMEM_SHARED((_G * _V, _D), jnp.float32),
            pltpu.VMEM((_RPW,), jnp.int32),
            pltpu.VMEM((2, _CHUNK, _D), jnp.float32),
            pltpu.SemaphoreType.DMA,
            pltpu.SemaphoreType.DMA,
            pltpu.SemaphoreType.DMA,
            pltpu.SemaphoreType.DMA,
        ],
    )
    def _sc_gather(cv_hbm, idx_hbm, out_hbm, stage_v, shared, idx_v, rows_v,
                   g0, g1, s0, s1):
        sid = lax.axis_index("s")
        wid = sid * _NC + lax.axis_index("c")
        base = wid * _RPW

        @pl.when(sid == 0)
        def _():
            pltpu.sync_copy(cv_hbm, stage_v)
            pltpu.sync_copy(stage_v, shared)

        pltpu.sync_copy(idx_hbm.at[pl.ds(base, _RPW)], idx_v)
        plsc.subcore_barrier()

        gsem = [g0, g1]
        ssem = [s0, s1]

        def gather(c):
            return pltpu.async_copy(
                shared.at[idx_v.at[pl.ds(c * _CHUNK, _CHUNK)]],
                rows_v.at[c % 2], gsem[c % 2])

        def store(c):
            return pltpu.async_copy(
                rows_v.at[c % 2], out_hbm.at[pl.ds(base + c * _CHUNK, _CHUNK)],
                ssem[c % 2])

        prev_g = gather(0)
        prev_s = None
        for c in range(_NCH):
            prev_g.wait()
            if prev_s is not None:
                prev_s.wait()
            if c + 1 < _NCH:
                prev_g = gather(c + 1)
            prev_s = store(c)
        prev_s.wait()

    return _sc_gather


@jax.jit
def kernel(hidden_states, W, b, codevectors):
    x = hidden_states.reshape(_T, _H)
    b2 = b.reshape(_G * _V, 1)
    idx, perp = pl.pallas_call(
        _tc_body,
        in_specs=[
            pl.BlockSpec(memory_space=pl.ANY),
            pl.BlockSpec((_G * _V, _H), lambda: (0, 0)),
            pl.BlockSpec((_G * _V, 1), lambda: (0, 0)),
        ],
        out_specs=[
            pl.BlockSpec((_T, _G), lambda: (0, 0)),
            pl.BlockSpec((1, 1), lambda: (0, 0)),
        ],
        out_shape=[
            jax.ShapeDtypeStruct((_T, _G), jnp.int32),
            jax.ShapeDtypeStruct((1, 1), jnp.float32),
        ],
        scratch_shapes=[
            pltpu.VMEM((2, _TB, _H), jnp.float32),
            pltpu.SemaphoreType.DMA,
            pltpu.SemaphoreType.DMA,
        ],
    )(x, W, b2)
    cv_flat = codevectors.reshape(_G * _V, _D)
    out = _make_sc_gather()(cv_flat, idx.reshape(_ROWS))
    return out.reshape(_B, _S, _G * _D), perp[0, 0]


# speedup vs baseline: 1.0051x; 1.0051x over previous
import functools

import jax
import jax.numpy as jnp
from jax import lax
from jax.experimental import pallas as pl
from jax.experimental.pallas import tpu as pltpu
from jax.experimental.pallas import tpu_sc as plsc

_B, _S, _H = 8, 2048, 512
_G, _V = 2, 320
_D = 128
_T = _B * _S
_TB = 2048
_NBLK = _T // _TB

_NC, _NS = 2, 16
_NW = _NC * _NS
_ROWS = _T * _G
_RPW = _ROWS // _NW
_CHUNK = 128
_NCH = _RPW // _CHUNK


_LW = 128


def _lane_fold(t):
    w = t.shape[1]
    while w > _LW:
        w //= 2
        t = t[:, :w] + t[:, w:]
    return t


def _argmax_cols(lt):
    m = jnp.max(lt, axis=0, keepdims=True)
    eq = lt == m
    iota = lax.broadcasted_iota(jnp.int32, lt.shape, 0)
    idx = jnp.min(jnp.where(eq, iota, _V), axis=0, keepdims=True)
    cnt = _lane_fold(eq.astype(jnp.float32))
    return idx, cnt


_SPLIT = 4
_SB = _TB // _SPLIT


_NSTR = 4
_SR = _TB // _NSTR


def _tc_body(x_hbm, w_ref, b_ref, idx_ref, perp_ref, xbuf, s0, s1):
    sems = [s0, s1]

    class stream:
        def __init__(self, k):
            self.copies = [
                pltpu.make_async_copy(
                    x_hbm.at[pl.ds(k * _TB + j * _SR, _SR), :],
                    xbuf.at[k % 2, pl.ds(j * _SR, _SR), :],
                    sems[k % 2])
                for j in range(_NSTR)]

        def start(self):
            for c in self.copies:
                c.start()

        def wait(self):
            for c in self.copies:
                c.wait()

    stream(0).start()
    c0 = jnp.zeros((_V, _LW), jnp.float32)
    c1 = jnp.zeros((_V, _LW), jnp.float32)
    for blk in range(_NBLK):
        if blk + 1 < _NBLK:
            stream(blk + 1).start()
        stream(blk).wait()
        parts = []
        for k in range(_SPLIT):
            lt = lax.dot_general(
                w_ref[...], xbuf[blk % 2, pl.ds(k * _SB, _SB), :],
                (((1,), (1,)), ((), ())),
                preferred_element_type=jnp.float32,
                precision=lax.Precision.DEFAULT,
            ) + b_ref[...]
            idx0, p0 = _argmax_cols(lt[:_V])
            idx1, p1 = _argmax_cols(lt[_V:])
            parts.append(jnp.concatenate([idx0, idx1 + _V], axis=0))
            c0 = c0 + p0
            c1 = c1 + p1
        idx_t = jnp.concatenate(parts, axis=1)
        idx_ref[pl.ds(blk * _TB, _TB), :] = jnp.transpose(idx_t, (1, 0))

    p0 = jnp.sum(c0, axis=1, keepdims=True) * (1.0 / _T)
    p1 = jnp.sum(c1, axis=1, keepdims=True) * (1.0 / _T)
    e0 = -jnp.sum(p0 * jnp.log(p0 + 1e-7), keepdims=True)
    e1 = -jnp.sum(p1 * jnp.log(p1 + 1e-7), keepdims=True)
    perp_ref[...] = jnp.exp(e0) + jnp.exp(e1)


@functools.lru_cache(maxsize=1)
def _make_sc_gather():
    @functools.partial(
        pl.kernel,
        mesh=plsc.VectorSubcoreMesh(core_axis_name="c", subcore_axis_name="s"),
        out_type=jax.ShapeDtypeStruct((_ROWS, _D), jnp.float32),
        scratch_types=[
            pltpu.VMEM((_G * _V, _D), jnp.float32),
            pltpu.VMEM_SHARED((_G * _V, _D), jnp.float32),
            pltpu.VMEM((_RPW,), jnp.int32),
            pltpu.VMEM((2, _CHUNK, _D), jnp.float32),
            pltpu.SemaphoreType.DMA,
            pltpu.SemaphoreType.DMA,
            pltpu.SemaphoreType.DMA,
            pltpu.SemaphoreType.DMA,
        ],
    )
    def _sc_gather(cv_hbm, idx_hbm, out_hbm, stage_v, shared, idx_v, rows_v,
                   g0, g1, s0, s1):
        sid = lax.axis_index("s")
        wid = sid * _NC + lax.axis_index("c")
        base = wid * _RPW

        @pl.when(sid == 0)
        def _():
            pltpu.sync_copy(cv_hbm, stage_v)
            pltpu.sync_copy(stage_v, shared)

        pltpu.sync_copy(idx_hbm.at[pl.ds(base, _RPW)], idx_v)
        plsc.subcore_barrier()

        gsem = [g0, g1]
        ssem = [s0, s1]

        def gather(c):
            return pltpu.async_copy(
                shared.at[idx_v.at[pl.ds(c * _CHUNK, _CHUNK)]],
                rows_v.at[c % 2], gsem[c % 2])

        def store(c):
            return pltpu.async_copy(
                rows_v.at[c % 2], out_hbm.at[pl.ds(base + c * _CHUNK, _CHUNK)],
                ssem[c % 2])

        prev_g = gather(0)
        prev_s = None
        for c in range(_NCH):
            prev_g.wait()
            if prev_s is not None:
                prev_s.wait()
            if c + 1 < _NCH:
                prev_g = gather(c + 1)
            prev_s = store(c)
        prev_s.wait()

    return _sc_gather


@jax.jit
def kernel(hidden_states, W, b, codevectors):
    x = hidden_states.reshape(_T, _H)
    b2 = b.reshape(_G * _V, 1)
    idx, perp = pl.pallas_call(
        _tc_body,
        in_specs=[
            pl.BlockSpec(memory_space=pl.ANY),
            pl.BlockSpec((_G * _V, _H), lambda: (0, 0)),
            pl.BlockSpec((_G * _V, 1), lambda: (0, 0)),
        ],
        out_specs=[
            pl.BlockSpec((_T, _G), lambda: (0, 0)),
            pl.BlockSpec((1, 1), lambda: (0, 0)),
        ],
        out_shape=[
            jax.ShapeDtypeStruct((_T, _G), jnp.int32),
            jax.ShapeDtypeStruct((1, 1), jnp.float32),
        ],
        scratch_shapes=[
            pltpu.VMEM((2, _TB, _H), jnp.float32),
            pltpu.SemaphoreType.DMA,
            pltpu.SemaphoreType.DMA,
        ],
    )(x, W, b2)
    cv_flat = codevectors.reshape(_G * _V, _D)
    out = _make_sc_gather()(cv_flat, idx.reshape(_ROWS))
    return out.reshape(_B, _S, _G * _D), perp[0, 0]


# speedup vs baseline: 1.0140x; 1.0088x over previous
import functools

import jax
import jax.numpy as jnp
from jax import lax
from jax.experimental import pallas as pl
from jax.experimental.pallas import tpu as pltpu
from jax.experimental.pallas import tpu_sc as plsc

_B, _S, _H = 8, 2048, 512
_G, _V = 2, 320
_D = 128
_T = _B * _S
_TB = 2048
_NBLK = _T // _TB

_NC, _NS = 2, 16
_NW = _NC * _NS
_ROWS = _T * _G
_RPW = _ROWS // _NW
_CHUNK = 128
_NCH = _RPW // _CHUNK


_LW = 128


def _lane_fold(t):
    w = t.shape[1]
    while w > _LW:
        w //= 2
        t = t[:, :w] + t[:, w:]
    return t


def _argmax_cols(lt):
    m = jnp.max(lt, axis=0, keepdims=True)
    eq = lt == m
    iota = lax.broadcasted_iota(jnp.int32, lt.shape, 0)
    idx = jnp.min(jnp.where(eq, iota, _V), axis=0, keepdims=True)
    cnt = _lane_fold(eq.astype(jnp.float32))
    return idx, cnt


_SPLIT = 4
_SB = _TB // _SPLIT


def _tc_body(x_ref, w_ref, b_ref, idx_ref, perp_ref, acc0, acc1):
    i = pl.program_id(0)
    parts = []
    c0 = c1 = None
    for k in range(_SPLIT):
        lt = lax.dot_general(
            w_ref[...], x_ref[pl.ds(k * _SB, _SB), :],
            (((1,), (1,)), ((), ())),
            preferred_element_type=jnp.float32,
            precision=lax.Precision.DEFAULT,
        ) + b_ref[...]
        idx0, p0 = _argmax_cols(lt[:_V])
        idx1, p1 = _argmax_cols(lt[_V:])
        parts.append(jnp.concatenate([idx0, idx1 + _V], axis=0))
        c0 = p0 if c0 is None else c0 + p0
        c1 = p1 if c1 is None else c1 + p1
    idx_t = jnp.concatenate(parts, axis=1)
    idx_ref[...] = jnp.transpose(idx_t, (1, 0))

    @pl.when(i == 0)
    def _():
        acc0[...] = c0
        acc1[...] = c1

    @pl.when(i > 0)
    def _():
        acc0[...] += c0
        acc1[...] += c1

    @pl.when(i == _NBLK - 1)
    def _():
        p0 = jnp.sum(acc0[...], axis=1, keepdims=True) * (1.0 / _T)
        p1 = jnp.sum(acc1[...], axis=1, keepdims=True) * (1.0 / _T)
        e0 = -jnp.sum(p0 * jnp.log(p0 + 1e-7), keepdims=True)
        e1 = -jnp.sum(p1 * jnp.log(p1 + 1e-7), keepdims=True)
        perp_ref[...] = jnp.exp(e0) + jnp.exp(e1)


@functools.lru_cache(maxsize=1)
def _make_sc_gather():
    @functools.partial(
        pl.kernel,
        mesh=plsc.VectorSubcoreMesh(core_axis_name="c", subcore_axis_name="s"),
        out_type=jax.ShapeDtypeStruct((_ROWS, _D), jnp.float32),
        scratch_types=[
            pltpu.VMEM_SHARED((_G * _V, _D), jnp.float32),
            pltpu.VMEM((_RPW,), jnp.int32),
            pltpu.VMEM((2, 2 * _CHUNK, _D), jnp.float32),
            pltpu.SemaphoreType.DMA,
            pltpu.SemaphoreType.DMA,
            pltpu.SemaphoreType.DMA,
            pltpu.SemaphoreType.DMA,
        ],
    )
    def _sc_gather(cv_hbm, idx_hbm, out_hbm, shared, idx_v, rows_v,
                   g0, g1, s0, s1):
        sid = lax.axis_index("s")
        wid = sid * _NC + lax.axis_index("c")
        base = wid * _RPW

        @pl.when(sid == 0)
        def _():
            for j, n in ((0, 256), (256, 256), (512, 128)):
                pltpu.sync_copy(cv_hbm.at[pl.ds(j, n)],
                                rows_v.at[0, pl.ds(0, n), :])
                pltpu.sync_copy(rows_v.at[0, pl.ds(0, n), :],
                                shared.at[pl.ds(j, n)])

        pltpu.sync_copy(idx_hbm.at[pl.ds(base, _RPW)], idx_v)
        plsc.subcore_barrier()

        gsem = [g0, g1]
        ssem = [s0, s1]
        nsup = _NCH // 2

        def gather(u, half):
            c = 2 * u + half
            return pltpu.async_copy(
                shared.at[idx_v.at[pl.ds(c * _CHUNK, _CHUNK)]],
                rows_v.at[u % 2, pl.ds(half * _CHUNK, _CHUNK), :],
                gsem[u % 2])

        def store(u):
            return pltpu.async_copy(
                rows_v.at[u % 2],
                out_hbm.at[pl.ds(base + 2 * u * _CHUNK, 2 * _CHUNK)],
                ssem[u % 2])

        prev_g = (gather(0, 0), gather(0, 1))
        prev_s = None
        for u in range(nsup):
            prev_g[0].wait()
            prev_g[1].wait()
            if prev_s is not None:
                prev_s.wait()
            if u + 1 < nsup:
                prev_g = (gather(u + 1, 0), gather(u + 1, 1))
            prev_s = store(u)
        prev_s.wait()

    return _sc_gather


@jax.jit
def kernel(hidden_states, W, b, codevectors):
    x = hidden_states.reshape(_T, _H)
    b2 = b.reshape(_G * _V, 1)
    idx, perp = pl.pallas_call(
        _tc_body,
        grid=(_NBLK,),
        in_specs=[
            pl.BlockSpec((_TB, _H), lambda i: (i, 0)),
            pl.BlockSpec((_G * _V, _H), lambda i: (0, 0)),
            pl.BlockSpec((_G * _V, 1), lambda i: (0, 0)),
        ],
        out_specs=[
            pl.BlockSpec((_TB, _G), lambda i: (i, 0)),
            pl.BlockSpec((1, 1), lambda i: (0, 0)),
        ],
        out_shape=[
            jax.ShapeDtypeStruct((_T, _G), jnp.int32),
            jax.ShapeDtypeStruct((1, 1), jnp.float32),
        ],
        scratch_shapes=[
            pltpu.VMEM((_V, _LW), jnp.float32),
            pltpu.VMEM((_V, _LW), jnp.float32),
        ],
    )(x, W, b2)
    cv_flat = codevectors.reshape(_G * _V, _D)
    out = _make_sc_gather()(cv_flat, idx.reshape(_ROWS))
    return out.reshape(_B, _S, _G * _D), perp[0, 0]


# speedup vs baseline: 1.0304x; 1.0162x over previous
import functools

import jax
import jax.numpy as jnp
from jax import lax
from jax.experimental import pallas as pl
from jax.experimental.pallas import tpu as pltpu
from jax.experimental.pallas import tpu_sc as plsc

_B, _S, _H = 8, 2048, 512
_G, _V = 2, 320
_D = 128
_T = _B * _S
_TB = 2048
_NBLK = _T // _TB

_NC, _NS = 2, 16
_NW = _NC * _NS
_ROWS = _T * _G
_RPW = _ROWS // _NW
_CHUNK = 128
_NCH = _RPW // _CHUNK


_LW = 128


def _lane_fold(t):
    w = t.shape[1]
    while w > _LW:
        w //= 2
        t = t[:, :w] + t[:, w:]
    return t


def _argmax_cols(lt):
    m = jnp.max(lt, axis=0, keepdims=True)
    eq = lt == m
    iota = lax.broadcasted_iota(jnp.int32, lt.shape, 0)
    idx = jnp.min(jnp.where(eq, iota, _V), axis=0, keepdims=True)
    cnt = _lane_fold(eq.astype(jnp.float32))
    return idx, cnt


_SPLIT = 4
_SB = _TB // _SPLIT


def _tc_body(x_ref, w_ref, b_ref, idx_ref, perp_ref, acc0, acc1):
    i = pl.program_id(0)
    parts = []
    c0 = c1 = None
    for k in range(_SPLIT):
        lt = lax.dot_general(
            w_ref[...], x_ref[pl.ds(k * _SB, _SB), :],
            (((1,), (1,)), ((), ())),
            preferred_element_type=jnp.float32,
            precision=lax.Precision.DEFAULT,
        ) + b_ref[...]
        idx0, p0 = _argmax_cols(lt[:_V])
        idx1, p1 = _argmax_cols(lt[_V:])
        parts.append(jnp.concatenate([idx0, idx1 + _V], axis=0))
        c0 = p0 if c0 is None else c0 + p0
        c1 = p1 if c1 is None else c1 + p1
    idx_t = jnp.concatenate(parts, axis=1)
    idx_ref[...] = jnp.transpose(idx_t, (1, 0))

    @pl.when(i == 0)
    def _():
        acc0[...] = c0
        acc1[...] = c1

    @pl.when(i > 0)
    def _():
        acc0[...] += c0
        acc1[...] += c1

    @pl.when(i == _NBLK - 1)
    def _():
        p0 = jnp.sum(acc0[...], axis=1, keepdims=True) * (1.0 / _T)
        p1 = jnp.sum(acc1[...], axis=1, keepdims=True) * (1.0 / _T)
        e0 = -jnp.sum(p0 * jnp.log(p0 + 1e-7), keepdims=True)
        e1 = -jnp.sum(p1 * jnp.log(p1 + 1e-7), keepdims=True)
        perp_ref[...] = jnp.exp(e0) + jnp.exp(e1)


@functools.lru_cache(maxsize=1)
def _make_sc_gather():
    @functools.partial(
        pl.kernel,
        mesh=plsc.VectorSubcoreMesh(core_axis_name="c", subcore_axis_name="s"),
        out_type=jax.ShapeDtypeStruct((_ROWS, _D), jnp.float32),
        scratch_types=[
            pltpu.VMEM((_G * _V, _D), jnp.float32),
            pltpu.VMEM_SHARED((_G * _V, _D), jnp.float32),
            pltpu.VMEM((_RPW,), jnp.int32),
            pltpu.VMEM((2, _CHUNK, _D), jnp.float32),
            pltpu.SemaphoreType.DMA,
            pltpu.SemaphoreType.DMA,
            pltpu.SemaphoreType.DMA,
            pltpu.SemaphoreType.DMA,
        ],
    )
    def _sc_gather(cv_hbm, idx_hbm, out_hbm, stage_v, shared, idx_v, rows_v,
                   g0, g1, s0, s1):
        sid = lax.axis_index("s")
        wid = sid * _NC + lax.axis_index("c")
        base = wid * _RPW

        @pl.when(sid == 0)
        def _():
            pltpu.sync_copy(cv_hbm, stage_v)
            pltpu.sync_copy(stage_v, shared)

        pltpu.sync_copy(idx_hbm.at[pl.ds(base, _RPW)], idx_v)
        plsc.subcore_barrier()

        gsem = [g0, g1]
        ssem = [s0, s1]

        def gather(c):
            return pltpu.async_copy(
                shared.at[idx_v.at[pl.ds(c * _CHUNK, _CHUNK)]],
                rows_v.at[c % 2], gsem[c % 2])

        def store(c):
            return pltpu.async_copy(
                rows_v.at[c % 2], out_hbm.at[pl.ds(base + c * _CHUNK, _CHUNK)],
                ssem[c % 2])

        prev_g = gather(0)
        prev_s = None
        for c in range(_NCH):
            prev_g.wait()
            if prev_s is not None:
                prev_s.wait()
            if c + 1 < _NCH:
                prev_g = gather(c + 1)
            prev_s = store(c)
        prev_s.wait()

    return _sc_gather


@jax.jit
def kernel(hidden_states, W, b, codevectors):
    x = hidden_states.reshape(_T, _H)
    b2 = b.reshape(_G * _V, 1)
    idx, perp = pl.pallas_call(
        _tc_body,
        grid=(_NBLK,),
        in_specs=[
            pl.BlockSpec((_TB, _H), lambda i: (i, 0)),
            pl.BlockSpec((_G * _V, _H), lambda i: (0, 0)),
            pl.BlockSpec((_G * _V, 1), lambda i: (0, 0)),
        ],
        out_specs=[
            pl.BlockSpec((_TB, _G), lambda i: (i, 0)),
            pl.BlockSpec((1, 1), lambda i: (0, 0)),
        ],
        out_shape=[
            jax.ShapeDtypeStruct((_T, _G), jnp.int32),
            jax.ShapeDtypeStruct((1, 1), jnp.float32),
        ],
        scratch_shapes=[
            pltpu.VMEM((_V, _LW), jnp.float32),
            pltpu.VMEM((_V, _LW), jnp.float32),
        ],
    )(x, W, b2)
    cv_flat = codevectors.reshape(_G * _V, _D)
    out = _make_sc_gather()(cv_flat, idx.reshape(_ROWS))
    return out.reshape(_B, _S, _G * _D), perp[0, 0]
